# baseline (speedup 1.0000x reference)
import jax
import jax.numpy as jnp
from jax import lax
from jax.experimental import pallas as pl
from jax.experimental.pallas import tpu as pltpu
from jax.experimental.pallas import tpu_sc as plsc

_B, _S, _D, _F, _E = 2, 4096, 768, 1024, 16
_N = _B * _S
_THRESH = 0.8


_TG = 1024


def _gate_body(x_ref, w_ref, b_ref, o_ref, aa_ref):
  i = pl.program_id(0)
  logits = (
      jnp.dot(x_ref[...], w_ref[...], preferred_element_type=jnp.float32)
      + b_ref[...]
  )
  o_ref[...] = logits

  @pl.when(i == 0)
  def _init():
    aa_ref[...] = jnp.zeros((1, _E), jnp.float32)

  rowmax = jnp.max(logits, axis=1, keepdims=True)
  has0 = jnp.max((logits[:, 0:1] == rowmax).astype(jnp.float32))

  @pl.when(has0 > 0.0)
  def _top1_path():
    one0 = (lax.broadcasted_iota(jnp.int32, (1, _E), 1) == 0)
    aa_ref[...] = jnp.maximum(aa_ref[...], one0.astype(jnp.float32))

  @pl.when(has0 == 0.0)
  def _exact_path():
    ex = jnp.exp(logits)
    z = jnp.sum(ex, axis=1)
    acc = jnp.zeros((1, _E), jnp.float32)
    lane = lax.broadcasted_iota(jnp.int32, (_TG, _E), 1)
    for e in range(_E):
      ecol = ex[:, e:e + 1]
      bigger = (ex > ecol) | ((ex == ecol) & (lane < e))
      s = jnp.sum(jnp.where(bigger, ex, 0.0), axis=1)
      any_e = jnp.max((s < _THRESH * z).astype(jnp.float32))
      acc = jnp.where(lax.broadcasted_iota(jnp.int32, (1, _E), 1) == e,
                      any_e, acc)
    aa_ref[...] = jnp.maximum(aa_ref[...], acc)


def _gate(x_flat, gate_w, gate_b):
  return pl.pallas_call(
      _gate_body,
      grid=(_N // _TG,),
      in_specs=[
          pl.BlockSpec((_TG, _D), lambda i: (i, 0)),
          pl.BlockSpec((_D, _E), lambda i: (0, 0)),
          pl.BlockSpec((1, _E), lambda i: (0, 0)),
      ],
      out_specs=[
          pl.BlockSpec((_TG, _E), lambda i: (i, 0)),
          pl.BlockSpec((1, _E), lambda i: (0, 0)),
      ],
      out_shape=[
          jax.ShapeDtypeStruct((_N, _E), jnp.float32),
          jax.ShapeDtypeStruct((1, _E), jnp.float32),
      ],
  )(x_flat, gate_w, gate_b.reshape(1, _E))


try:
  _INFO = plsc.get_sparse_core_info()
  _NC, _NS, _L = _INFO.num_cores, _INFO.num_subcores, _INFO.num_lanes
except ValueError:
  _NC, _NS, _L = 2, 16, 16
_NW = _NC * _NS
_TPW = _N // _NW


def _router_body(logits_hbm, rw_hbm, probs_hbm, ac_hbm, log_v, rw_v, p_v,
                 ac_v):
  c = lax.axis_index("c")
  s = lax.axis_index("s")
  wid = s * _NC + c
  base = wid * _TPW
  pltpu.sync_copy(logits_hbm.at[pl.ds(base, _TPW), :], log_v)
  eidx = lax.iota(jnp.int32, _L)

  @plsc.parallel_loop(0, _TPW // _L, unroll=4)
  def group(g):
    acc = jnp.zeros((_L,), jnp.int32)
    for j in range(_L):
      i = g * _L + j
      lv = log_v[i, :]
      ex = jnp.exp(lv)
      z = jnp.sum(ex)
      p_v[i, :] = ex / z
      es, order = plsc.sort_key_val(ex, eidx, descending=True)
      shifted = plsc.cumsum(es) - es
      act = shifted < _THRESH * z
      ap = jnp.where(act, es, jnp.zeros_like(es))
      aw = ap / (jnp.sum(ap) + 1e-6 * z)
      plsc.store_scatter(rw_v.at[i], [order], aw)
      acc = jnp.where(eidx == j, plsc.all_reduce_population_count(act), acc)
    ac_v[pl.ds(g * _L, _L)] = acc

  pltpu.sync_copy(rw_v, rw_hbm.at[pl.ds(base, _TPW), :])
  pltpu.sync_copy(p_v, probs_hbm.at[pl.ds(base, _TPW), :])
  pltpu.sync_copy(ac_v, ac_hbm.at[pl.ds(base, _TPW)])


def _router(logits):
  f32 = jnp.float32
  return pl.kernel(
      _router_body,
      out_type=(
          jax.ShapeDtypeStruct((_N, _E), f32),
          jax.ShapeDtypeStruct((_N, _E), f32),
          jax.ShapeDtypeStruct((_N,), jnp.int32),
      ),
      mesh=plsc.VectorSubcoreMesh(
          core_axis_name="c", subcore_axis_name="s"
      ),
      compiler_params=pltpu.CompilerParams(needs_layout_passes=False),
      scratch_types=[
          pltpu.VMEM((_TPW, _E), f32),
          pltpu.VMEM((_TPW, _E), f32),
          pltpu.VMEM((_TPW, _E), f32),
          pltpu.VMEM((_TPW,), jnp.int32),
      ],
  )(logits)


_TT = 2048


def _first_from(aa_vec):
  cand = jnp.where(aa_vec > 0.0, lax.iota(jnp.int32, _E), _E)
  fm = jnp.min(cand)
  return jnp.where(fm == _E, 0, fm)


def _mlp_body(aa_ref, x_ref, w1_hbm, w3_hbm, w2_hbm, o_ref,
              w1_v, w3_v, w2_v, sems):
  i = pl.program_id(0)

  @pl.when(i == 0)
  def _prologue():
    first = _first_from(aa_ref[0, :])
    pltpu.make_async_copy(w1_hbm.at[first], w1_v, sems.at[0]).start()
    pltpu.make_async_copy(w3_hbm.at[first], w3_v, sems.at[1]).start()
    pltpu.make_async_copy(w2_hbm.at[first], w2_v, sems.at[2]).start()
    pltpu.make_async_copy(w1_hbm.at[first], w1_v, sems.at[0]).wait()
    pltpu.make_async_copy(w3_hbm.at[first], w3_v, sems.at[1]).wait()
    pltpu.make_async_copy(w2_hbm.at[first], w2_v, sems.at[2]).wait()

  xb = x_ref[...]
  h1 = jnp.dot(xb, w1_v[...], preferred_element_type=jnp.float32)
  h3 = jnp.dot(xb, w3_v[...], preferred_element_type=jnp.float32)
  h = h1 * jax.nn.sigmoid(h1) * h3
  o_ref[...] = jnp.dot(
      h, w2_v[...], preferred_element_type=jnp.float32
  ).astype(jnp.bfloat16)


def _mlp(aa, x_flat, w1, w3, w2):
  return pl.pallas_call(
      _mlp_body,
      grid=(_N // _TT,),
      in_specs=[
          pl.BlockSpec((1, _E), lambda i: (0, 0)),
          pl.BlockSpec((_TT, _D), lambda i: (i, 0)),
          pl.BlockSpec(memory_space=pl.ANY),
          pl.BlockSpec(memory_space=pl.ANY),
          pl.BlockSpec(memory_space=pl.ANY),
      ],
      out_specs=pl.BlockSpec((_TT, _D), lambda i: (i, 0)),
      out_shape=jax.ShapeDtypeStruct((_N, _D), jnp.bfloat16),
      scratch_shapes=[
          pltpu.VMEM((_D, _F), jnp.float32),
          pltpu.VMEM((_D, _F), jnp.float32),
          pltpu.VMEM((_F, _D), jnp.float32),
          pltpu.SemaphoreType.DMA((3,)),
      ],
  )(aa, x_flat, w1, w3, w2)


_TL = 2048


def _scale_body(aa_ref, rw_ref, p_ref, u_ref, o_ref, lb_ref, le_ref,
                first_sm, ent_sm, tpe_v, psum_v):
  i = pl.program_id(0)
  nsteps = pl.num_programs(0)

  @pl.when(i == 0)
  def _prologue():
    first_sm[0] = _first_from(aa_ref[0, :])
    ent_sm[0] = 0.0
    tpe_v[...] = jnp.zeros((1, _E), jnp.float32)
    psum_v[...] = jnp.zeros((1, _E), jnp.float32)

  rw = rw_ref[...]
  p = p_ref[...]
  mask = (rw > 0.0).astype(jnp.float32)
  tpe_v[...] += jnp.sum(mask, axis=0, keepdims=True)
  psum_v[...] += jnp.sum(p, axis=0, keepdims=True)
  ent_sm[0] += jnp.sum(p * jnp.log(p + 1e-6))

  lane = lax.broadcasted_iota(jnp.int32, (_TL, _E), 1)
  scale = jnp.sum(
      jnp.where(lane == first_sm[0], rw, 0.0), axis=1, keepdims=True)
  o_ref[...] = u_ref[...].astype(jnp.float32) * scale

  @pl.when(i == nsteps - 1)
  def _epilogue():
    lb = _E * jnp.sum((tpe_v[0, :] / _N) * (psum_v[0, :] / _N))
    lb_ref[...] = jnp.full((1, 1), lb, jnp.float32)
    le_ref[...] = jnp.full((1, 1), -ent_sm[0] / _N, jnp.float32)


def _scale(aa, rw, probs, u):
  return pl.pallas_call(
      _scale_body,
      grid=(_N // _TL,),
      in_specs=[
          pl.BlockSpec((1, _E), lambda i: (0, 0)),
          pl.BlockSpec((_TL, _E), lambda i: (i, 0)),
          pl.BlockSpec((_TL, _E), lambda i: (i, 0)),
          pl.BlockSpec((_TL, _D), lambda i: (i, 0)),
      ],
      out_specs=[
          pl.BlockSpec((_TL, _D), lambda i: (i, 0)),
          pl.BlockSpec((1, 1), lambda i: (0, 0)),
          pl.BlockSpec((1, 1), lambda i: (0, 0)),
      ],
      out_shape=[
          jax.ShapeDtypeStruct((_N, _D), jnp.float32),
          jax.ShapeDtypeStruct((1, 1), jnp.float32),
          jax.ShapeDtypeStruct((1, 1), jnp.float32),
      ],
      scratch_shapes=[
          pltpu.SMEM((1,), jnp.int32),
          pltpu.SMEM((1,), jnp.float32),
          pltpu.VMEM((1, _E), jnp.float32),
          pltpu.VMEM((1, _E), jnp.float32),
      ],
  )(aa, rw, probs, u)


@jax.jit
def kernel(x, gate_w, gate_b, w1, w3, w2):
  x_flat = x.reshape(_N, _D)
  logits, aa = _gate(x_flat, gate_w, gate_b)
  rw, probs, ac = _router(logits)
  u = _mlp(aa, x_flat, w1, w3, w2)
  out, lb, le = _scale(aa, rw, probs, u)
  return (
      out.reshape(_B, _S, _D),
      lb.reshape(()),
      le.reshape(()),
      ac.reshape(_B, _S),
  )


# speedup vs baseline: 1.0095x; 1.0095x over previous
import jax
import jax.numpy as jnp
from jax import lax
from jax.experimental import pallas as pl
from jax.experimental.pallas import tpu as pltpu
from jax.experimental.pallas import tpu_sc as plsc

_B, _S, _D, _F, _E = 2, 4096, 768, 1024, 16
_N = _B * _S
_THRESH = 0.8


_TG = 1024


def _gate_body(x_ref, w_ref, b_ref, o_ref, aa_ref):
  i = pl.program_id(0)
  logits = (
      jnp.dot(x_ref[...], w_ref[...], preferred_element_type=jnp.float32)
      + b_ref[...]
  )
  o_ref[...] = logits

  @pl.when(i == 0)
  def _init():
    aa_ref[...] = jnp.zeros((1, _E), jnp.float32)

  rowmax = jnp.max(logits, axis=1, keepdims=True)
  has0 = jnp.max((logits[:, 0:1] == rowmax).astype(jnp.float32))

  @pl.when(has0 > 0.0)
  def _top1_path():
    one0 = (lax.broadcasted_iota(jnp.int32, (1, _E), 1) == 0)
    aa_ref[...] = jnp.maximum(aa_ref[...], one0.astype(jnp.float32))

  @pl.when(has0 == 0.0)
  def _exact_path():
    ex = jnp.exp(logits)
    z = jnp.sum(ex, axis=1)
    acc = jnp.zeros((1, _E), jnp.float32)
    lane = lax.broadcasted_iota(jnp.int32, (_TG, _E), 1)
    for e in range(_E):
      ecol = ex[:, e:e + 1]
      bigger = (ex > ecol) | ((ex == ecol) & (lane < e))
      s = jnp.sum(jnp.where(bigger, ex, 0.0), axis=1)
      any_e = jnp.max((s < _THRESH * z).astype(jnp.float32))
      acc = jnp.where(lax.broadcasted_iota(jnp.int32, (1, _E), 1) == e,
                      any_e, acc)
    aa_ref[...] = jnp.maximum(aa_ref[...], acc)


def _gate(x_flat, gate_w, gate_b):
  return pl.pallas_call(
      _gate_body,
      grid=(_N // _TG,),
      in_specs=[
          pl.BlockSpec((_TG, _D), lambda i: (i, 0)),
          pl.BlockSpec((_D, _E), lambda i: (0, 0)),
          pl.BlockSpec((1, _E), lambda i: (0, 0)),
      ],
      out_specs=[
          pl.BlockSpec((_TG, _E), lambda i: (i, 0)),
          pl.BlockSpec((1, _E), lambda i: (0, 0)),
      ],
      out_shape=[
          jax.ShapeDtypeStruct((_N, _E), jnp.float32),
          jax.ShapeDtypeStruct((1, _E), jnp.float32),
      ],
  )(x_flat, gate_w, gate_b.reshape(1, _E))


try:
  _INFO = plsc.get_sparse_core_info()
  _NC, _NS, _L = _INFO.num_cores, _INFO.num_subcores, _INFO.num_lanes
except ValueError:
  _NC, _NS, _L = 2, 16, 16
_NW = _NC * _NS
_TPW = _N // _NW


def _router_body(logits_hbm, rw_hbm, probs_hbm, ac_hbm, log_v, rw_v, p_v,
                 ac_v):
  c = lax.axis_index("c")
  s = lax.axis_index("s")
  wid = s * _NC + c
  base = wid * _TPW
  pltpu.sync_copy(logits_hbm.at[pl.ds(base, _TPW), :], log_v)
  eidx = lax.iota(jnp.int32, _L)

  @plsc.parallel_loop(0, _TPW // _L, unroll=4)
  def group(g):
    acc = jnp.zeros((_L,), jnp.int32)
    for j in range(_L):
      i = g * _L + j
      lv = log_v[i, :]
      ex = jnp.exp(lv)
      z = jnp.sum(ex)
      p_v[i, :] = ex / z
      es, order = plsc.sort_key_val(ex, eidx, descending=True)
      shifted = plsc.cumsum(es) - es
      act = shifted < _THRESH * z
      ap = jnp.where(act, es, jnp.zeros_like(es))
      aw = ap / (jnp.sum(ap) + 1e-6 * z)
      plsc.store_scatter(rw_v.at[i], [order], aw)
      acc = jnp.where(eidx == j, plsc.all_reduce_population_count(act), acc)
    ac_v[pl.ds(g * _L, _L)] = acc

  pltpu.sync_copy(rw_v, rw_hbm.at[pl.ds(base, _TPW), :])
  pltpu.sync_copy(p_v, probs_hbm.at[pl.ds(base, _TPW), :])
  pltpu.sync_copy(ac_v, ac_hbm.at[pl.ds(base, _TPW)])


def _router(logits):
  f32 = jnp.float32
  return pl.kernel(
      _router_body,
      out_type=(
          jax.ShapeDtypeStruct((_N, _E), f32),
          jax.ShapeDtypeStruct((_N, _E), f32),
          jax.ShapeDtypeStruct((_N,), jnp.int32),
      ),
      mesh=plsc.VectorSubcoreMesh(
          core_axis_name="c", subcore_axis_name="s"
      ),
      compiler_params=pltpu.CompilerParams(needs_layout_passes=False),
      scratch_types=[
          pltpu.VMEM((_TPW, _E), f32),
          pltpu.VMEM((_TPW, _E), f32),
          pltpu.VMEM((_TPW, _E), f32),
          pltpu.VMEM((_TPW,), jnp.int32),
      ],
  )(logits)


_TT = 1024


def _first_from(aa_vec):
  cand = jnp.where(aa_vec > 0.0, lax.iota(jnp.int32, _E), _E)
  fm = jnp.min(cand)
  return jnp.where(fm == _E, 0, fm)


def _mlp_body(aa_ref, x_ref, w1_hbm, w3_hbm, w2_hbm, o_ref,
              w1_v, w3_v, w2_v, sems):
  i = pl.program_id(0)

  @pl.when(i == 0)
  def _prologue():
    first = _first_from(aa_ref[0, :])
    pltpu.make_async_copy(w1_hbm.at[first], w1_v, sems.at[0]).start()
    pltpu.make_async_copy(w3_hbm.at[first], w3_v, sems.at[1]).start()
    pltpu.make_async_copy(w2_hbm.at[first], w2_v, sems.at[2]).start()
    pltpu.make_async_copy(w1_hbm.at[first], w1_v, sems.at[0]).wait()
    pltpu.make_async_copy(w3_hbm.at[first], w3_v, sems.at[1]).wait()
    pltpu.make_async_copy(w2_hbm.at[first], w2_v, sems.at[2]).wait()

  xb = x_ref[...]
  h1 = jnp.dot(xb, w1_v[...], preferred_element_type=jnp.float32)
  h3 = jnp.dot(xb, w3_v[...], preferred_element_type=jnp.float32)
  h = h1 * jax.nn.sigmoid(h1) * h3
  o_ref[...] = jnp.dot(
      h, w2_v[...], preferred_element_type=jnp.float32
  ).astype(jnp.bfloat16)


def _mlp(aa, x_flat, w1, w3, w2):
  return pl.pallas_call(
      _mlp_body,
      grid=(_N // _TT,),
      in_specs=[
          pl.BlockSpec((1, _E), lambda i: (0, 0)),
          pl.BlockSpec((_TT, _D), lambda i: (i, 0)),
          pl.BlockSpec(memory_space=pl.ANY),
          pl.BlockSpec(memory_space=pl.ANY),
          pl.BlockSpec(memory_space=pl.ANY),
      ],
      out_specs=pl.BlockSpec((_TT, _D), lambda i: (i, 0)),
      out_shape=jax.ShapeDtypeStruct((_N, _D), jnp.bfloat16),
      scratch_shapes=[
          pltpu.VMEM((_D, _F), jnp.float32),
          pltpu.VMEM((_D, _F), jnp.float32),
          pltpu.VMEM((_F, _D), jnp.float32),
          pltpu.SemaphoreType.DMA((3,)),
      ],
  )(aa, x_flat, w1, w3, w2)


_TL = 2048


def _scale_body(aa_ref, rw_ref, p_ref, u_ref, o_ref, lb_ref, le_ref,
                first_sm, ent_sm, tpe_v, psum_v):
  i = pl.program_id(0)
  nsteps = pl.num_programs(0)

  @pl.when(i == 0)
  def _prologue():
    first_sm[0] = _first_from(aa_ref[0, :])
    ent_sm[0] = 0.0
    tpe_v[...] = jnp.zeros((1, _E), jnp.float32)
    psum_v[...] = jnp.zeros((1, _E), jnp.float32)

  rw = rw_ref[...]
  p = p_ref[...]
  mask = (rw > 0.0).astype(jnp.float32)
  tpe_v[...] += jnp.sum(mask, axis=0, keepdims=True)
  psum_v[...] += jnp.sum(p, axis=0, keepdims=True)
  ent_sm[0] += jnp.sum(p * jnp.log(p + 1e-6))

  lane = lax.broadcasted_iota(jnp.int32, (_TL, _E), 1)
  scale = jnp.sum(
      jnp.where(lane == first_sm[0], rw, 0.0), axis=1, keepdims=True)
  o_ref[...] = u_ref[...].astype(jnp.float32) * scale

  @pl.when(i == nsteps - 1)
  def _epilogue():
    lb = _E * jnp.sum((tpe_v[0, :] / _N) * (psum_v[0, :] / _N))
    lb_ref[...] = jnp.full((1, 1), lb, jnp.float32)
    le_ref[...] = jnp.full((1, 1), -ent_sm[0] / _N, jnp.float32)


def _scale(aa, rw, probs, u):
  return pl.pallas_call(
      _scale_body,
      grid=(_N // _TL,),
      in_specs=[
          pl.BlockSpec((1, _E), lambda i: (0, 0)),
          pl.BlockSpec((_TL, _E), lambda i: (i, 0)),
          pl.BlockSpec((_TL, _E), lambda i: (i, 0)),
          pl.BlockSpec((_TL, _D), lambda i: (i, 0)),
      ],
      out_specs=[
          pl.BlockSpec((_TL, _D), lambda i: (i, 0)),
          pl.BlockSpec((1, 1), lambda i: (0, 0)),
          pl.BlockSpec((1, 1), lambda i: (0, 0)),
      ],
      out_shape=[
          jax.ShapeDtypeStruct((_N, _D), jnp.float32),
          jax.ShapeDtypeStruct((1, 1), jnp.float32),
          jax.ShapeDtypeStruct((1, 1), jnp.float32),
      ],
      scratch_shapes=[
          pltpu.SMEM((1,), jnp.int32),
          pltpu.SMEM((1,), jnp.float32),
          pltpu.VMEM((1, _E), jnp.float32),
          pltpu.VMEM((1, _E), jnp.float32),
      ],
  )(aa, rw, probs, u)


@jax.jit
def kernel(x, gate_w, gate_b, w1, w3, w2):
  x_flat = x.reshape(_N, _D)
  logits, aa = _gate(x_flat, gate_w, gate_b)
  rw, probs, ac = _router(logits)
  u = _mlp(aa, x_flat, w1, w3, w2)
  out, lb, le = _scale(aa, rw, probs, u)
  return (
      out.reshape(_B, _S, _D),
      lb.reshape(()),
      le.reshape(()),
      ac.reshape(_B, _S),
  )


# speedup vs baseline: 1.0327x; 1.0230x over previous
import jax
import jax.numpy as jnp
from jax import lax
from jax.experimental import pallas as pl
from jax.experimental.pallas import tpu as pltpu
from jax.experimental.pallas import tpu_sc as plsc

_B, _S, _D, _F, _E = 2, 4096, 768, 1024, 16
_N = _B * _S
_THRESH = 0.8


_TG = 2048


def _gate_body(x_ref, w_ref, b_ref, o_ref, aa_ref):
  i = pl.program_id(0)
  logits = (
      jnp.dot(x_ref[...], w_ref[...], preferred_element_type=jnp.float32)
      + b_ref[...]
  )
  o_ref[...] = logits

  @pl.when(i == 0)
  def _init():
    aa_ref[...] = jnp.zeros((1, _E), jnp.float32)

  rowmax = jnp.max(logits, axis=1, keepdims=True)
  has0 = jnp.max((logits[:, 0:1] == rowmax).astype(jnp.float32))

  @pl.when(has0 > 0.0)
  def _top1_path():
    one0 = (lax.broadcasted_iota(jnp.int32, (1, _E), 1) == 0)
    aa_ref[...] = jnp.maximum(aa_ref[...], one0.astype(jnp.float32))

  @pl.when(has0 == 0.0)
  def _exact_path():
    ex = jnp.exp(logits)
    z = jnp.sum(ex, axis=1)
    acc = jnp.zeros((1, _E), jnp.float32)
    lane = lax.broadcasted_iota(jnp.int32, (_TG, _E), 1)
    for e in range(_E):
      ecol = ex[:, e:e + 1]
      bigger = (ex > ecol) | ((ex == ecol) & (lane < e))
      s = jnp.sum(jnp.where(bigger, ex, 0.0), axis=1)
      any_e = jnp.max((s < _THRESH * z).astype(jnp.float32))
      acc = jnp.where(lax.broadcasted_iota(jnp.int32, (1, _E), 1) == e,
                      any_e, acc)
    aa_ref[...] = jnp.maximum(aa_ref[...], acc)


def _gate(x_flat, gate_w, gate_b):
  return pl.pallas_call(
      _gate_body,
      grid=(_N // _TG,),
      in_specs=[
          pl.BlockSpec((_TG, _D), lambda i: (i, 0)),
          pl.BlockSpec((_D, _E), lambda i: (0, 0)),
          pl.BlockSpec((1, _E), lambda i: (0, 0)),
      ],
      out_specs=[
          pl.BlockSpec((_TG, _E), lambda i: (i, 0)),
          pl.BlockSpec((1, _E), lambda i: (0, 0)),
      ],
      out_shape=[
          jax.ShapeDtypeStruct((_N, _E), jnp.float32),
          jax.ShapeDtypeStruct((1, _E), jnp.float32),
      ],
  )(x_flat, gate_w, gate_b.reshape(1, _E))


try:
  _INFO = plsc.get_sparse_core_info()
  _NC, _NS, _L = _INFO.num_cores, _INFO.num_subcores, _INFO.num_lanes
except ValueError:
  _NC, _NS, _L = 2, 16, 16
_NW = _NC * _NS
_TPW = _N // _NW


def _router_body(logits_hbm, rw_hbm, probs_hbm, ac_hbm, log_v, rw_v, p_v,
                 ac_v):
  c = lax.axis_index("c")
  s = lax.axis_index("s")
  wid = s * _NC + c
  base = wid * _TPW
  pltpu.sync_copy(logits_hbm.at[pl.ds(base, _TPW), :], log_v)
  eidx = lax.iota(jnp.int32, _L)

  @plsc.parallel_loop(0, _TPW // _L, unroll=4)
  def group(g):
    acc = jnp.zeros((_L,), jnp.int32)
    for j in range(_L):
      i = g * _L + j
      lv = log_v[i, :]
      ex = jnp.exp(lv)
      z = jnp.sum(ex)
      p_v[i, :] = ex / z
      es, order = plsc.sort_key_val(ex, eidx, descending=True)
      shifted = plsc.cumsum(es) - es
      act = shifted < _THRESH * z
      ap = jnp.where(act, es, jnp.zeros_like(es))
      aw = ap / (jnp.sum(ap) + 1e-6 * z)
      plsc.store_scatter(rw_v.at[i], [order], aw)
      acc = jnp.where(eidx == j, plsc.all_reduce_population_count(act), acc)
    ac_v[pl.ds(g * _L, _L)] = acc

  pltpu.sync_copy(rw_v, rw_hbm.at[pl.ds(base, _TPW), :])
  pltpu.sync_copy(p_v, probs_hbm.at[pl.ds(base, _TPW), :])
  pltpu.sync_copy(ac_v, ac_hbm.at[pl.ds(base, _TPW)])


def _router(logits):
  f32 = jnp.float32
  return pl.kernel(
      _router_body,
      out_type=(
          jax.ShapeDtypeStruct((_N, _E), f32),
          jax.ShapeDtypeStruct((_N, _E), f32),
          jax.ShapeDtypeStruct((_N,), jnp.int32),
      ),
      mesh=plsc.VectorSubcoreMesh(
          core_axis_name="c", subcore_axis_name="s"
      ),
      compiler_params=pltpu.CompilerParams(needs_layout_passes=False),
      scratch_types=[
          pltpu.VMEM((_TPW, _E), f32),
          pltpu.VMEM((_TPW, _E), f32),
          pltpu.VMEM((_TPW, _E), f32),
          pltpu.VMEM((_TPW,), jnp.int32),
      ],
  )(logits)


_TT = 1024


def _first_from(aa_vec):
  cand = jnp.where(aa_vec > 0.0, lax.iota(jnp.int32, _E), _E)
  fm = jnp.min(cand)
  return jnp.where(fm == _E, 0, fm)


def _mlp_body(aa_ref, x_ref, w1_hbm, w3_hbm, w2_hbm, o_ref,
              w1_v, w3_v, w2_v, sems):
  i = pl.program_id(0)

  @pl.when(i == 0)
  def _prologue():
    first = _first_from(aa_ref[0, :])
    pltpu.make_async_copy(w1_hbm.at[first], w1_v, sems.at[0]).start()
    pltpu.make_async_copy(w3_hbm.at[first], w3_v, sems.at[1]).start()
    pltpu.make_async_copy(w2_hbm.at[first], w2_v, sems.at[2]).start()
    pltpu.make_async_copy(w1_hbm.at[first], w1_v, sems.at[0]).wait()
    pltpu.make_async_copy(w3_hbm.at[first], w3_v, sems.at[1]).wait()
    pltpu.make_async_copy(w2_hbm.at[first], w2_v, sems.at[2]).wait()

  xb = x_ref[...]
  h1 = jnp.dot(xb, w1_v[...], preferred_element_type=jnp.float32)
  h3 = jnp.dot(xb, w3_v[...], preferred_element_type=jnp.float32)
  h = h1 * jax.nn.sigmoid(h1) * h3
  o_ref[...] = jnp.dot(
      h, w2_v[...], preferred_element_type=jnp.float32
  ).astype(jnp.bfloat16)


def _mlp(aa, x_flat, w1, w3, w2):
  return pl.pallas_call(
      _mlp_body,
      grid=(_N // _TT,),
      in_specs=[
          pl.BlockSpec((1, _E), lambda i: (0, 0)),
          pl.BlockSpec((_TT, _D), lambda i: (i, 0)),
          pl.BlockSpec(memory_space=pl.ANY),
          pl.BlockSpec(memory_space=pl.ANY),
          pl.BlockSpec(memory_space=pl.ANY),
      ],
      out_specs=pl.BlockSpec((_TT, _D), lambda i: (i, 0)),
      out_shape=jax.ShapeDtypeStruct((_N, _D), jnp.bfloat16),
      scratch_shapes=[
          pltpu.VMEM((_D, _F), jnp.float32),
          pltpu.VMEM((_D, _F), jnp.float32),
          pltpu.VMEM((_F, _D), jnp.float32),
          pltpu.SemaphoreType.DMA((3,)),
      ],
  )(aa, x_flat, w1, w3, w2)


_TL = 2048


def _scale_body(aa_ref, rw_ref, p_ref, u_ref, o_ref, lb_ref, le_ref,
                first_sm, ent_sm, tpe_v, psum_v):
  i = pl.program_id(0)
  nsteps = pl.num_programs(0)

  @pl.when(i == 0)
  def _prologue():
    first_sm[0] = _first_from(aa_ref[0, :])
    ent_sm[0] = 0.0
    tpe_v[...] = jnp.zeros((1, _E), jnp.float32)
    psum_v[...] = jnp.zeros((1, _E), jnp.float32)

  rw = rw_ref[...]
  p = p_ref[...]
  mask = (rw > 0.0).astype(jnp.float32)
  tpe_v[...] += jnp.sum(mask, axis=0, keepdims=True)
  psum_v[...] += jnp.sum(p, axis=0, keepdims=True)
  ent_sm[0] += jnp.sum(p * jnp.log(p + 1e-6))

  lane = lax.broadcasted_iota(jnp.int32, (_TL, _E), 1)
  scale = jnp.sum(
      jnp.where(lane == first_sm[0], rw, 0.0), axis=1, keepdims=True)
  o_ref[...] = u_ref[...].astype(jnp.float32) * scale

  @pl.when(i == nsteps - 1)
  def _epilogue():
    lb = _E * jnp.sum((tpe_v[0, :] / _N) * (psum_v[0, :] / _N))
    lb_ref[...] = jnp.full((1, 1), lb, jnp.float32)
    le_ref[...] = jnp.full((1, 1), -ent_sm[0] / _N, jnp.float32)


def _scale(aa, rw, probs, u):
  return pl.pallas_call(
      _scale_body,
      grid=(_N // _TL,),
      in_specs=[
          pl.BlockSpec((1, _E), lambda i: (0, 0)),
          pl.BlockSpec((_TL, _E), lambda i: (i, 0)),
          pl.BlockSpec((_TL, _E), lambda i: (i, 0)),
          pl.BlockSpec((_TL, _D), lambda i: (i, 0)),
      ],
      out_specs=[
          pl.BlockSpec((_TL, _D), lambda i: (i, 0)),
          pl.BlockSpec((1, 1), lambda i: (0, 0)),
          pl.BlockSpec((1, 1), lambda i: (0, 0)),
      ],
      out_shape=[
          jax.ShapeDtypeStruct((_N, _D), jnp.float32),
          jax.ShapeDtypeStruct((1, 1), jnp.float32),
          jax.ShapeDtypeStruct((1, 1), jnp.float32),
      ],
      scratch_shapes=[
          pltpu.SMEM((1,), jnp.int32),
          pltpu.SMEM((1,), jnp.float32),
          pltpu.VMEM((1, _E), jnp.float32),
          pltpu.VMEM((1, _E), jnp.float32),
      ],
  )(aa, rw, probs, u)


@jax.jit
def kernel(x, gate_w, gate_b, w1, w3, w2):
  x_flat = x.reshape(_N, _D)
  logits, aa = _gate(x_flat, gate_w, gate_b)
  rw, probs, ac = _router(logits)
  u = _mlp(aa, x_flat, w1, w3, w2)
  out, lb, le = _scale(aa, rw, probs, u)
  return (
      out.reshape(_B, _S, _D),
      lb.reshape(()),
      le.reshape(()),
      ac.reshape(_B, _S),
  )


# speedup vs baseline: 1.0373x; 1.0045x over previous
import jax
import jax.numpy as jnp
from jax import lax
from jax.experimental import pallas as pl
from jax.experimental.pallas import tpu as pltpu
from jax.experimental.pallas import tpu_sc as plsc

_B, _S, _D, _F, _E = 2, 4096, 768, 1024, 16
_N = _B * _S
_THRESH = 0.8


_TG = 4096


def _gate_body(x_ref, w_ref, b_ref, o_ref, aa_ref):
  i = pl.program_id(0)
  logits = (
      jnp.dot(x_ref[...], w_ref[...], preferred_element_type=jnp.float32)
      + b_ref[...]
  )
  o_ref[...] = logits

  @pl.when(i == 0)
  def _init():
    aa_ref[...] = jnp.zeros((1, _E), jnp.float32)

  rowmax = jnp.max(logits, axis=1, keepdims=True)
  has0 = jnp.max((logits[:, 0:1] == rowmax).astype(jnp.float32))

  @pl.when(has0 > 0.0)
  def _top1_path():
    one0 = (lax.broadcasted_iota(jnp.int32, (1, _E), 1) == 0)
    aa_ref[...] = jnp.maximum(aa_ref[...], one0.astype(jnp.float32))

  @pl.when(has0 == 0.0)
  def _exact_path():
    ex = jnp.exp(logits)
    z = jnp.sum(ex, axis=1)
    acc = jnp.zeros((1, _E), jnp.float32)
    lane = lax.broadcasted_iota(jnp.int32, (_TG, _E), 1)
    for e in range(_E):
      ecol = ex[:, e:e + 1]
      bigger = (ex > ecol) | ((ex == ecol) & (lane < e))
      s = jnp.sum(jnp.where(bigger, ex, 0.0), axis=1)
      any_e = jnp.max((s < _THRESH * z).astype(jnp.float32))
      acc = jnp.where(lax.broadcasted_iota(jnp.int32, (1, _E), 1) == e,
                      any_e, acc)
    aa_ref[...] = jnp.maximum(aa_ref[...], acc)


def _gate(x_flat, gate_w, gate_b):
  return pl.pallas_call(
      _gate_body,
      grid=(_N // _TG,),
      in_specs=[
          pl.BlockSpec((_TG, _D), lambda i: (i, 0)),
          pl.BlockSpec((_D, _E), lambda i: (0, 0)),
          pl.BlockSpec((1, _E), lambda i: (0, 0)),
      ],
      out_specs=[
          pl.BlockSpec((_TG, _E), lambda i: (i, 0)),
          pl.BlockSpec((1, _E), lambda i: (0, 0)),
      ],
      out_shape=[
          jax.ShapeDtypeStruct((_N, _E), jnp.float32),
          jax.ShapeDtypeStruct((1, _E), jnp.float32),
      ],
  )(x_flat, gate_w, gate_b.reshape(1, _E))


try:
  _INFO = plsc.get_sparse_core_info()
  _NC, _NS, _L = _INFO.num_cores, _INFO.num_subcores, _INFO.num_lanes
except ValueError:
  _NC, _NS, _L = 2, 16, 16
_NW = _NC * _NS
_TPW = _N // _NW


def _router_body(logits_hbm, rw_hbm, probs_hbm, ac_hbm, log_v, rw_v, p_v,
                 ac_v):
  c = lax.axis_index("c")
  s = lax.axis_index("s")
  wid = s * _NC + c
  base = wid * _TPW
  pltpu.sync_copy(logits_hbm.at[pl.ds(base, _TPW), :], log_v)
  eidx = lax.iota(jnp.int32, _L)

  @plsc.parallel_loop(0, _TPW // _L, unroll=4)
  def group(g):
    acc = jnp.zeros((_L,), jnp.int32)
    for j in range(_L):
      i = g * _L + j
      lv = log_v[i, :]
      ex = jnp.exp(lv)
      z = jnp.sum(ex)
      p_v[i, :] = ex / z
      es, order = plsc.sort_key_val(ex, eidx, descending=True)
      shifted = plsc.cumsum(es) - es
      act = shifted < _THRESH * z
      ap = jnp.where(act, es, jnp.zeros_like(es))
      aw = ap / (jnp.sum(ap) + 1e-6 * z)
      plsc.store_scatter(rw_v.at[i], [order], aw)
      acc = jnp.where(eidx == j, plsc.all_reduce_population_count(act), acc)
    ac_v[pl.ds(g * _L, _L)] = acc

  pltpu.sync_copy(rw_v, rw_hbm.at[pl.ds(base, _TPW), :])
  pltpu.sync_copy(p_v, probs_hbm.at[pl.ds(base, _TPW), :])
  pltpu.sync_copy(ac_v, ac_hbm.at[pl.ds(base, _TPW)])


def _router(logits):
  f32 = jnp.float32
  return pl.kernel(
      _router_body,
      out_type=(
          jax.ShapeDtypeStruct((_N, _E), f32),
          jax.ShapeDtypeStruct((_N, _E), f32),
          jax.ShapeDtypeStruct((_N,), jnp.int32),
      ),
      mesh=plsc.VectorSubcoreMesh(
          core_axis_name="c", subcore_axis_name="s"
      ),
      compiler_params=pltpu.CompilerParams(needs_layout_passes=False),
      scratch_types=[
          pltpu.VMEM((_TPW, _E), f32),
          pltpu.VMEM((_TPW, _E), f32),
          pltpu.VMEM((_TPW, _E), f32),
          pltpu.VMEM((_TPW,), jnp.int32),
      ],
  )(logits)


_TT = 1024


def _first_from(aa_vec):
  cand = jnp.where(aa_vec > 0.0, lax.iota(jnp.int32, _E), _E)
  fm = jnp.min(cand)
  return jnp.where(fm == _E, 0, fm)


def _mlp_body(aa_ref, x_ref, w1_hbm, w3_hbm, w2_hbm, o_ref,
              w1_v, w3_v, w2_v, sems):
  i = pl.program_id(0)

  @pl.when(i == 0)
  def _prologue():
    first = _first_from(aa_ref[0, :])
    pltpu.make_async_copy(w1_hbm.at[first], w1_v, sems.at[0]).start()
    pltpu.make_async_copy(w3_hbm.at[first], w3_v, sems.at[1]).start()
    pltpu.make_async_copy(w2_hbm.at[first], w2_v, sems.at[2]).start()
    pltpu.make_async_copy(w1_hbm.at[first], w1_v, sems.at[0]).wait()
    pltpu.make_async_copy(w3_hbm.at[first], w3_v, sems.at[1]).wait()
    pltpu.make_async_copy(w2_hbm.at[first], w2_v, sems.at[2]).wait()

  xb = x_ref[...]
  h1 = jnp.dot(xb, w1_v[...], preferred_element_type=jnp.float32)
  h3 = jnp.dot(xb, w3_v[...], preferred_element_type=jnp.float32)
  h = h1 * jax.nn.sigmoid(h1) * h3
  o_ref[...] = jnp.dot(
      h, w2_v[...], preferred_element_type=jnp.float32
  ).astype(jnp.bfloat16)


def _mlp(aa, x_flat, w1, w3, w2):
  return pl.pallas_call(
      _mlp_body,
      grid=(_N // _TT,),
      in_specs=[
          pl.BlockSpec((1, _E), lambda i: (0, 0)),
          pl.BlockSpec((_TT, _D), lambda i: (i, 0)),
          pl.BlockSpec(memory_space=pl.ANY),
          pl.BlockSpec(memory_space=pl.ANY),
          pl.BlockSpec(memory_space=pl.ANY),
      ],
      out_specs=pl.BlockSpec((_TT, _D), lambda i: (i, 0)),
      out_shape=jax.ShapeDtypeStruct((_N, _D), jnp.bfloat16),
      scratch_shapes=[
          pltpu.VMEM((_D, _F), jnp.float32),
          pltpu.VMEM((_D, _F), jnp.float32),
          pltpu.VMEM((_F, _D), jnp.float32),
          pltpu.SemaphoreType.DMA((3,)),
      ],
  )(aa, x_flat, w1, w3, w2)


_TL = 2048


def _scale_body(aa_ref, rw_ref, p_ref, u_ref, o_ref, lb_ref, le_ref,
                first_sm, ent_sm, tpe_v, psum_v):
  i = pl.program_id(0)
  nsteps = pl.num_programs(0)

  @pl.when(i == 0)
  def _prologue():
    first_sm[0] = _first_from(aa_ref[0, :])
    ent_sm[0] = 0.0
    tpe_v[...] = jnp.zeros((1, _E), jnp.float32)
    psum_v[...] = jnp.zeros((1, _E), jnp.float32)

  rw = rw_ref[...]
  p = p_ref[...]
  mask = (rw > 0.0).astype(jnp.float32)
  tpe_v[...] += jnp.sum(mask, axis=0, keepdims=True)
  psum_v[...] += jnp.sum(p, axis=0, keepdims=True)
  ent_sm[0] += jnp.sum(p * jnp.log(p + 1e-6))

  lane = lax.broadcasted_iota(jnp.int32, (_TL, _E), 1)
  scale = jnp.sum(
      jnp.where(lane == first_sm[0], rw, 0.0), axis=1, keepdims=True)
  o_ref[...] = u_ref[...].astype(jnp.float32) * scale

  @pl.when(i == nsteps - 1)
  def _epilogue():
    lb = _E * jnp.sum((tpe_v[0, :] / _N) * (psum_v[0, :] / _N))
    lb_ref[...] = jnp.full((1, 1), lb, jnp.float32)
    le_ref[...] = jnp.full((1, 1), -ent_sm[0] / _N, jnp.float32)


def _scale(aa, rw, probs, u):
  return pl.pallas_call(
      _scale_body,
      grid=(_N // _TL,),
      in_specs=[
          pl.BlockSpec((1, _E), lambda i: (0, 0)),
          pl.BlockSpec((_TL, _E), lambda i: (i, 0)),
          pl.BlockSpec((_TL, _E), lambda i: (i, 0)),
          pl.BlockSpec((_TL, _D), lambda i: (i, 0)),
      ],
      out_specs=[
          pl.BlockSpec((_TL, _D), lambda i: (i, 0)),
          pl.BlockSpec((1, 1), lambda i: (0, 0)),
          pl.BlockSpec((1, 1), lambda i: (0, 0)),
      ],
      out_shape=[
          jax.ShapeDtypeStruct((_N, _D), jnp.float32),
          jax.ShapeDtypeStruct((1, 1), jnp.float32),
          jax.ShapeDtypeStruct((1, 1), jnp.float32),
      ],
      scratch_shapes=[
          pltpu.SMEM((1,), jnp.int32),
          pltpu.SMEM((1,), jnp.float32),
          pltpu.VMEM((1, _E), jnp.float32),
          pltpu.VMEM((1, _E), jnp.float32),
      ],
  )(aa, rw, probs, u)


@jax.jit
def kernel(x, gate_w, gate_b, w1, w3, w2):
  x_flat = x.reshape(_N, _D)
  logits, aa = _gate(x_flat, gate_w, gate_b)
  rw, probs, ac = _router(logits)
  u = _mlp(aa, x_flat, w1, w3, w2)
  out, lb, le = _scale(aa, rw, probs, u)
  return (
      out.reshape(_B, _S, _D),
      lb.reshape(()),
      le.reshape(()),
      ac.reshape(_B, _S),
  )
